# baseline (speedup 1.0000x reference)
import functools

import jax
import jax.numpy as jnp
from jax import lax
from jax.experimental import pallas as pl
from jax.experimental.pallas import tpu as pltpu
from jax.experimental.pallas import tpu_sc as plsc

W = 4096
F = 64
D = 8
K = 2 * D + 1
B = 8
WP = W + 2 * D

NUM_WORKERS = 32
ROWS_PER_WORKER = (B * W) // NUM_WORKERS
WORKERS_PER_BATCH = W // ROWS_PER_WORKER
CH = 256
NCHUNK = ROWS_PER_WORKER // CH
NBUF = 2

DELAYS = [0] + [i for i in range(1, D + 1)] + [-i for i in range(1, D + 1)]

LANES = 16
FGROUPS = F // LANES


def _sc_body(subed_hbm, subpad_hbm, out_hbm, sub_bufs, subed_bufs, out_bufs,
             sems_in, sems_out):
    wid = lax.axis_index("s") * 2 + lax.axis_index("c")
    b = wid // WORKERS_PER_BATCH
    q = wid % WORKERS_PER_BATCH
    w0 = q * ROWS_PER_WORKER
    row0 = b * W + w0
    pad0 = b * WP + w0

    def in_copies(c, p):
        src0 = (pad0 + c * CH) * F
        dst0 = (row0 + c * CH) * F
        return (
            pltpu.make_async_copy(
                subpad_hbm.at[pl.ds(src0, (CH + 2 * D) * F)], sub_bufs[p],
                sems_in.at[p, 0]),
            pltpu.make_async_copy(
                subed_hbm.at[pl.ds(dst0, CH * F)], subed_bufs[p],
                sems_in.at[p, 1]),
        )

    def out_copy(c, p):
        dst0 = (row0 + c * CH) * F
        return pltpu.make_async_copy(
            out_bufs[p], out_hbm.at[pl.ds(dst0, CH * F)], sems_out.at[p])

    for c in range(NBUF):
        for cp in in_copies(c, c % NBUF):
            cp.start()

    for c in range(NCHUNK):
        p = c % NBUF
        for cp in in_copies(c, p):
            cp.wait()
        if c >= NBUF:
            out_copy(c - NBUF, p).wait()

        sub_buf = sub_bufs[p]
        subed_buf = subed_bufs[p]
        out_buf = out_bufs[p]

        def row_body(i, _):
            for f in range(FGROUPS):
                x = subed_buf[pl.ds(i * F + f * LANES, LANES)]
                best = x - sub_buf[pl.ds((i + D) * F + f * LANES, LANES)]
                besta = jnp.abs(best)
                for d in DELAYS[1:]:
                    r = x - sub_buf[pl.ds((i + D + d) * F + f * LANES, LANES)]
                    ra = jnp.abs(r)
                    m = ra < besta
                    best = jnp.where(m, r, best)
                    besta = jnp.where(m, ra, besta)
                out_buf[pl.ds(i * F + f * LANES, LANES)] = best
            return 0

        lax.fori_loop(0, CH, row_body, 0, unroll=2)

        out_copy(c, p).start()
        if c + NBUF < NCHUNK:
            for cp in in_copies(c + NBUF, p):
                cp.start()

    for c in range(NCHUNK - NBUF, NCHUNK):
        out_copy(c, c % NBUF).wait()


@jax.jit
def kernel(subed, sub):
    sub_pad = jnp.pad(sub, ((0, 0), (D, D), (0, 0)), mode="edge")
    subed_flat = subed.reshape(B * W * F)
    subpad_flat = sub_pad.reshape(B * WP * F)

    mesh = plsc.VectorSubcoreMesh(core_axis_name="c", subcore_axis_name="s")
    out = pl.kernel(
        _sc_body,
        out_type=jax.ShapeDtypeStruct((B * W * F,), jnp.float32),
        mesh=mesh,
        scratch_types=[
            [pltpu.VMEM(((CH + 2 * D) * F,), jnp.float32) for _ in range(NBUF)],
            [pltpu.VMEM((CH * F,), jnp.float32) for _ in range(NBUF)],
            [pltpu.VMEM((CH * F,), jnp.float32) for _ in range(NBUF)],
            pltpu.SemaphoreType.DMA((NBUF, 2)),
            pltpu.SemaphoreType.DMA((NBUF,)),
        ],
    )(subed_flat, subpad_flat)
    return out.reshape(B, W, F)


# speedup vs baseline: 1.0273x; 1.0273x over previous
import functools

import jax
import jax.numpy as jnp
from jax import lax
from jax.experimental import pallas as pl
from jax.experimental.pallas import tpu as pltpu
from jax.experimental.pallas import tpu_sc as plsc

W = 4096
F = 64
D = 8
K = 2 * D + 1
B = 8

NUM_WORKERS = 32
ROWS_PER_WORKER = (B * W) // NUM_WORKERS
WORKERS_PER_BATCH = W // ROWS_PER_WORKER
CH = 256
NCHUNK = ROWS_PER_WORKER // CH
NBUF = 2

DELAYS = [0] + [i for i in range(1, D + 1)] + [-i for i in range(1, D + 1)]

LANES = 16
FGROUPS = F // LANES


def _sc_body(subed_hbm, sub_hbm, out_hbm, sub_bufs, subed_bufs, out_bufs,
             sems_in, sems_out):
    wid = lax.axis_index("s") * 2 + lax.axis_index("c")
    b = wid // WORKERS_PER_BATCH
    q = wid % WORKERS_PER_BATCH
    w0 = q * ROWS_PER_WORKER
    row0 = b * W + w0

    def sub_copy_mid(c, p):
        src0 = (row0 + c * CH - D) * F
        return pltpu.make_async_copy(
            sub_hbm.at[pl.ds(src0, (CH + 2 * D) * F)], sub_bufs[p],
            sems_in.at[p, 0])

    def sub_copy_first(p):
        return pltpu.make_async_copy(
            sub_hbm.at[pl.ds(row0 * F, (CH + D) * F)],
            sub_bufs[p].at[pl.ds(D * F, (CH + D) * F)], sems_in.at[p, 0])

    def sub_copy_last(c, p):
        src0 = (row0 + c * CH - D) * F
        return pltpu.make_async_copy(
            sub_hbm.at[pl.ds(src0, (CH + D) * F)],
            sub_bufs[p].at[pl.ds(0, (CH + D) * F)], sems_in.at[p, 0])

    def subed_copy(c, p):
        dst0 = (row0 + c * CH) * F
        return pltpu.make_async_copy(
            subed_hbm.at[pl.ds(dst0, CH * F)], subed_bufs[p],
            sems_in.at[p, 1])

    def start_in(c, p):
        subed_copy(c, p).start()
        if c == 0:
            @pl.when(q == 0)
            def _():
                sub_copy_first(p).start()

            @pl.when(q != 0)
            def _():
                sub_copy_mid(c, p).start()
        elif c == NCHUNK - 1:
            @pl.when(q == WORKERS_PER_BATCH - 1)
            def _():
                sub_copy_last(c, p).start()

            @pl.when(q != WORKERS_PER_BATCH - 1)
            def _():
                sub_copy_mid(c, p).start()
        else:
            sub_copy_mid(c, p).start()

    def wait_in(c, p):
        subed_copy(c, p).wait()
        if c == 0:
            @pl.when(q == 0)
            def _():
                sub_copy_first(p).wait()
                for f in range(FGROUPS):
                    v = sub_bufs[p][pl.ds(D * F + f * LANES, LANES)]
                    for t in range(D):
                        sub_bufs[p][pl.ds(t * F + f * LANES, LANES)] = v

            @pl.when(q != 0)
            def _():
                sub_copy_mid(c, p).wait()
        elif c == NCHUNK - 1:
            @pl.when(q == WORKERS_PER_BATCH - 1)
            def _():
                sub_copy_last(c, p).wait()
                for f in range(FGROUPS):
                    v = sub_bufs[p][pl.ds((CH + D - 1) * F + f * LANES, LANES)]
                    for t in range(CH + D, CH + 2 * D):
                        sub_bufs[p][pl.ds(t * F + f * LANES, LANES)] = v

            @pl.when(q != WORKERS_PER_BATCH - 1)
            def _():
                sub_copy_mid(c, p).wait()
        else:
            sub_copy_mid(c, p).wait()

    def out_copy(c, p):
        dst0 = (row0 + c * CH) * F
        return pltpu.make_async_copy(
            out_bufs[p], out_hbm.at[pl.ds(dst0, CH * F)], sems_out.at[p])

    for c in range(NBUF):
        start_in(c, c % NBUF)

    for c in range(NCHUNK):
        p = c % NBUF
        wait_in(c, p)
        if c >= NBUF:
            out_copy(c - NBUF, p).wait()

        sub_buf = sub_bufs[p]
        subed_buf = subed_bufs[p]
        out_buf = out_bufs[p]

        def row_body(i, _):
            for f in range(FGROUPS):
                x = subed_buf[pl.ds(i * F + f * LANES, LANES)]
                best = x - sub_buf[pl.ds((i + D) * F + f * LANES, LANES)]
                besta = jnp.abs(best)
                for d in DELAYS[1:]:
                    r = x - sub_buf[pl.ds((i + D + d) * F + f * LANES, LANES)]
                    ra = jnp.abs(r)
                    m = ra < besta
                    best = jnp.where(m, r, best)
                    besta = jnp.where(m, ra, besta)
                out_buf[pl.ds(i * F + f * LANES, LANES)] = best
            return 0

        lax.fori_loop(0, CH, row_body, 0, unroll=2)

        out_copy(c, p).start()
        if c + NBUF < NCHUNK:
            start_in(c + NBUF, p)

    for c in range(NCHUNK - NBUF, NCHUNK):
        out_copy(c, c % NBUF).wait()


@jax.jit
def kernel(subed, sub):
    subed_flat = subed.reshape(B * W * F)
    sub_flat = sub.reshape(B * W * F)

    mesh = plsc.VectorSubcoreMesh(core_axis_name="c", subcore_axis_name="s")
    out = pl.kernel(
        _sc_body,
        out_type=jax.ShapeDtypeStruct((B * W * F,), jnp.float32),
        mesh=mesh,
        scratch_types=[
            [pltpu.VMEM(((CH + 2 * D) * F,), jnp.float32) for _ in range(NBUF)],
            [pltpu.VMEM((CH * F,), jnp.float32) for _ in range(NBUF)],
            [pltpu.VMEM((CH * F,), jnp.float32) for _ in range(NBUF)],
            pltpu.SemaphoreType.DMA((NBUF, 2)),
            pltpu.SemaphoreType.DMA((NBUF,)),
        ],
    )(subed_flat, sub_flat)
    return out.reshape(B, W, F)


# speedup vs baseline: 1.4360x; 1.3978x over previous
import functools

import jax
import jax.numpy as jnp
from jax import lax
from jax.experimental import pallas as pl
from jax.experimental.pallas import tpu as pltpu
from jax.experimental.pallas import tpu_sc as plsc

W = 4096
F = 64
D = 8
K = 2 * D + 1
B = 8

NUM_WORKERS = 32
ROWS_PER_WORKER = (B * W) // NUM_WORKERS
WORKERS_PER_BATCH = W // ROWS_PER_WORKER
CH = 128
NCHUNK = ROWS_PER_WORKER // CH
NBUF = 2

DELAYS = [0] + [i for i in range(1, D + 1)] + [-i for i in range(1, D + 1)]

LANES = 16
FGROUPS = F // LANES


def _sc_body(subed_hbm, sub_hbm, out_hbm, sub_bufs, subed_bufs, out_bufs,
             sems_in, sems_out):
    wid = lax.axis_index("s") * 2 + lax.axis_index("c")
    b = wid // WORKERS_PER_BATCH
    q = wid % WORKERS_PER_BATCH
    w0 = q * ROWS_PER_WORKER
    row0 = b * W + w0

    first_q = q == 0
    last_q = q == WORKERS_PER_BATCH - 1

    def sub_copy_mid(c, p):
        return pltpu.make_async_copy(
            sub_hbm.at[pl.ds(row0 + c * CH - D, CH + 2 * D)], sub_bufs[p],
            sems_in.at[p, 0])

    def sub_copy_first(p):
        return pltpu.make_async_copy(
            sub_hbm.at[pl.ds(row0, CH + D)],
            sub_bufs[p].at[pl.ds(D, CH + D)], sems_in.at[p, 0])

    def sub_copy_last(c, p):
        return pltpu.make_async_copy(
            sub_hbm.at[pl.ds(row0 + c * CH - D, CH + D)],
            sub_bufs[p].at[pl.ds(0, CH + D)], sems_in.at[p, 0])

    def subed_copy(c, p):
        return pltpu.make_async_copy(
            subed_hbm.at[pl.ds(row0 + c * CH, CH)], subed_bufs[p],
            sems_in.at[p, 1])

    def start_in(c, p):
        subed_copy(c, p).start()
        is_first = jnp.logical_and(first_q, c == 0)
        is_last = jnp.logical_and(last_q, c == NCHUNK - 1)

        @pl.when(is_first)
        def _():
            sub_copy_first(p).start()

        @pl.when(is_last)
        def _():
            sub_copy_last(c, p).start()

        @pl.when(jnp.logical_not(jnp.logical_or(is_first, is_last)))
        def _():
            sub_copy_mid(c, p).start()

    def wait_in(c, p):
        subed_copy(c, p).wait()
        is_first = jnp.logical_and(first_q, c == 0)
        is_last = jnp.logical_and(last_q, c == NCHUNK - 1)

        @pl.when(is_first)
        def _():
            sub_copy_first(p).wait()
            for f in range(FGROUPS):
                fs = pl.ds(f * LANES, LANES)
                v = sub_bufs[p][D, fs]
                for t in range(D):
                    sub_bufs[p][t, fs] = v

        @pl.when(is_last)
        def _():
            sub_copy_last(c, p).wait()
            for f in range(FGROUPS):
                fs = pl.ds(f * LANES, LANES)
                v = sub_bufs[p][CH + D - 1, fs]
                for t in range(CH + D, CH + 2 * D):
                    sub_bufs[p][t, fs] = v

        @pl.when(jnp.logical_not(jnp.logical_or(is_first, is_last)))
        def _():
            sub_copy_mid(c, p).wait()

    def out_copy(c, p):
        return pltpu.make_async_copy(
            out_bufs[p], out_hbm.at[pl.ds(row0 + c * CH, CH)], sems_out.at[p])

    for p in range(NBUF):
        start_in(p, p)

    def pair_body(cc, _):
        for pp in range(NBUF):
            c = cc * NBUF + pp
            wait_in(c, pp)

            @pl.when(cc > 0)
            def _():
                out_copy(c - NBUF, pp).wait()

            sub_buf = sub_bufs[pp]
            subed_buf = subed_bufs[pp]
            out_buf = out_bufs[pp]

            def row_body(i, _):
                for f in range(FGROUPS):
                    fs = pl.ds(f * LANES, LANES)
                    x = subed_buf[i, fs]
                    best = x - sub_buf[i + D, fs]
                    besta = jnp.abs(best)
                    for d in DELAYS[1:]:
                        r = x - sub_buf[i + D + d, fs]
                        ra = jnp.abs(r)
                        m = ra < besta
                        best = jnp.where(m, r, best)
                        besta = jnp.where(m, ra, besta)
                    out_buf[i, fs] = best
                return 0

            lax.fori_loop(0, CH, row_body, 0)

            out_copy(c, pp).start()

            @pl.when(c + NBUF < NCHUNK)
            def _():
                start_in(c + NBUF, pp)
        return 0

    lax.fori_loop(0, NCHUNK // NBUF, pair_body, 0)

    for p in range(NBUF):
        out_copy(NCHUNK - NBUF + p, p).wait()


@jax.jit
def kernel(subed, sub):
    subed_flat = subed.reshape(B * W, F)
    sub_flat = sub.reshape(B * W, F)

    mesh = plsc.VectorSubcoreMesh(core_axis_name="c", subcore_axis_name="s")
    out = pl.kernel(
        _sc_body,
        out_type=jax.ShapeDtypeStruct((B * W, F), jnp.float32),
        mesh=mesh,
        scratch_types=[
            [pltpu.VMEM((CH + 2 * D, F), jnp.float32) for _ in range(NBUF)],
            [pltpu.VMEM((CH, F), jnp.float32) for _ in range(NBUF)],
            [pltpu.VMEM((CH, F), jnp.float32) for _ in range(NBUF)],
            pltpu.SemaphoreType.DMA((NBUF, 2)),
            pltpu.SemaphoreType.DMA((NBUF,)),
        ],
        compiler_params=pltpu.CompilerParams(use_tc_tiling_on_sc=True),
    )(subed_flat, sub_flat)
    return out.reshape(B, W, F)
